# baseline (speedup 1.0000x reference)
import functools

import jax
import jax.numpy as jnp
from jax import lax
from jax.experimental import pallas as pl
from jax.experimental.pallas import tpu as pltpu
from jax.experimental.pallas import tpu_sc as plsc

_B, _L, _C = 4096, 50, 16
_TOKEN_DIM, _CHAR_DIM = 64, 32
_CHAR_VOCAB = 128
_BL = _B * _L
_OUT_DIM = _TOKEN_DIM + _CHAR_DIM
_PAD_DIM = 128

_NC, _NS = 2, 16
_NW = _NC * _NS
_BPW = _BL // _NW
_CHUNK = 640
_NCHUNK = _BPW // _CHUNK
_GATHER = 128
_NGATHER = _CHUNK // _GATHER


def _sc_merge_body(table_hbm, idx_hbm, chars_hbm, out_hbm,
                   idx_v, rows_v, ch_v, sem):
    wid = lax.axis_index("s") * _NC + lax.axis_index("c")
    base = wid * _BPW
    pltpu.sync_copy(idx_hbm.at[pl.ds(base, _BPW)], idx_v)

    def step(m, carry):
        mb = m * _CHUNK
        copies = []
        for j in range(_NGATHER):
            copies.append(pltpu.async_copy(
                table_hbm.at[idx_v.at[pl.ds(mb + j * _GATHER, _GATHER)]],
                rows_v.at[pl.ds(j * _GATHER, _GATHER)],
                sem))
        copies.append(pltpu.async_copy(
            chars_hbm.at[pl.ds(base + mb, _CHUNK), pl.ds(0, _CHAR_DIM)],
            ch_v, sem))
        for cpy in copies:
            cpy.wait()
        pltpu.sync_copy(
            rows_v,
            out_hbm.at[pl.ds(base + mb, _CHUNK), pl.ds(0, _TOKEN_DIM)])
        pltpu.sync_copy(
            ch_v,
            out_hbm.at[pl.ds(base + mb, _CHUNK),
                       pl.ds(_TOKEN_DIM, _CHAR_DIM)])
        return carry

    lax.fori_loop(0, _NCHUNK, step, 0)


@jax.jit
def _sc_merge(token_table, flat_idx, chars_pad):
    mesh = plsc.VectorSubcoreMesh(core_axis_name="c", subcore_axis_name="s")
    return pl.kernel(
        _sc_merge_body,
        out_type=jax.ShapeDtypeStruct((_BL, _PAD_DIM), jnp.float32),
        mesh=mesh,
        scratch_types=[
            pltpu.VMEM((_BPW,), jnp.int32),
            pltpu.VMEM((_CHUNK, _TOKEN_DIM), jnp.float32),
            pltpu.VMEM((_CHUNK, _CHAR_DIM), jnp.float32),
            pltpu.SemaphoreType.DMA,
        ],
        compiler_params=pltpu.CompilerParams(use_tc_tiling_on_sc=False),
    )(token_table, flat_idx, chars_pad)


_P = _B


def _tc_chars_body(cs_ref, len_ref, tbl_ref, out_ref):
    cs = cs_ref[0]
    ln = jnp.maximum(len_ref[0], 1)
    valid = lax.broadcasted_iota(jnp.int32, (_C, _P), 0) < ln
    cs_m = jnp.where(valid, cs, _CHAR_VOCAB)
    inv_len = 1.0 / ln.astype(jnp.float32)
    vocab = lax.broadcasted_iota(jnp.int32, (_CHAR_VOCAB, _P), 0)
    acc = jnp.zeros((_CHAR_VOCAB, _P), jnp.float32)
    for c in range(_C):
        acc = acc + (cs_m[c:c + 1, :] == vocab).astype(jnp.float32)
    acc = acc * inv_len
    chars = lax.dot_general(acc, tbl_ref[...], (((0,), (0,)), ((), ())),
                            preferred_element_type=jnp.float32)
    out_ref[:, 0:_CHAR_DIM] = chars


@jax.jit
def _tc_chars(cs3, ln3, char_table):
    return pl.pallas_call(
        _tc_chars_body,
        grid=(_L,),
        in_specs=[
            pl.BlockSpec((1, _C, _P), lambda i: (i, 0, 0)),
            pl.BlockSpec((1, 1, _P), lambda i: (i, 0, 0)),
            pl.BlockSpec((_CHAR_VOCAB, _CHAR_DIM), lambda i: (0, 0)),
        ],
        out_specs=pl.BlockSpec((_P, _PAD_DIM), lambda i: (i, 0)),
        out_shape=jax.ShapeDtypeStruct((_BL, _PAD_DIM), jnp.float32),
    )(cs3, ln3, char_table)


def kernel(token_seq, char_seq, char_lengths, token_table, char_table):
    flat_idx = token_seq.T.reshape(_BL).astype(jnp.int32)
    cs3 = jnp.transpose(char_seq, (1, 2, 0)).astype(jnp.int32)
    ln3 = char_lengths.T.reshape(_L, 1, _B).astype(jnp.int32)
    half = token_table.shape[0] // 2
    h1 = lax.optimization_barrier(token_table[:half].T).T
    h2 = lax.optimization_barrier(token_table[half:].T).T
    tbl_rm = lax.concatenate([h1, h2], 0)
    chars_pad = _tc_chars(cs3, ln3, char_table)
    out_pad = _sc_merge(tbl_rm, flat_idx, chars_pad)
    out_lm = out_pad[:, :_OUT_DIM].reshape(_L, _B, _OUT_DIM)
    return jnp.transpose(out_lm, (1, 0, 2))


# speedup vs baseline: 2.0155x; 2.0155x over previous
import functools

import jax
import jax.numpy as jnp
from jax import lax
from jax.experimental import pallas as pl
from jax.experimental.pallas import tpu as pltpu
from jax.experimental.pallas import tpu_sc as plsc

_B, _L, _C = 4096, 50, 16
_TOKEN_DIM, _CHAR_DIM = 64, 32
_CHAR_VOCAB = 128
_BL = _B * _L
_OUT_DIM = _TOKEN_DIM + _CHAR_DIM
_PAD_DIM = 128

_NC, _NS = 2, 16
_NW = _NC * _NS
_BPW = _BL // _NW
_CHUNK = 640
_NCHUNK = _BPW // _CHUNK
_GATHER = 128
_NGATHER = _CHUNK // _GATHER


def _sc_merge_body(table_hbm, idx_hbm, chars_hbm, out_hbm,
                   idx_v, rows_v, ch_v, sem):
    wid = lax.axis_index("s") * _NC + lax.axis_index("c")
    base = wid * _BPW
    pltpu.sync_copy(idx_hbm.at[pl.ds(base, _BPW)], idx_v)

    def step(m, carry):
        mb = m * _CHUNK
        copies = []
        for j in range(_NGATHER):
            copies.append(pltpu.async_copy(
                table_hbm.at[idx_v.at[pl.ds(mb + j * _GATHER, _GATHER)]],
                rows_v.at[pl.ds(j * _GATHER, _GATHER)],
                sem))
        copies.append(pltpu.async_copy(
            chars_hbm.at[pl.ds(base + mb, _CHUNK), pl.ds(0, _CHAR_DIM)],
            ch_v, sem))
        for cpy in copies:
            cpy.wait()
        pltpu.sync_copy(
            rows_v,
            out_hbm.at[pl.ds(base + mb, _CHUNK), pl.ds(0, _TOKEN_DIM)])
        pltpu.sync_copy(
            ch_v,
            out_hbm.at[pl.ds(base + mb, _CHUNK),
                       pl.ds(_TOKEN_DIM, _CHAR_DIM)])
        return carry

    lax.fori_loop(0, _NCHUNK, step, 0)


@jax.jit
def _sc_merge(token_table, flat_idx, chars_pad):
    mesh = plsc.VectorSubcoreMesh(core_axis_name="c", subcore_axis_name="s")
    return pl.kernel(
        _sc_merge_body,
        out_type=jax.ShapeDtypeStruct((_BL, _PAD_DIM), jnp.float32),
        mesh=mesh,
        scratch_types=[
            pltpu.VMEM((_BPW,), jnp.int32),
            pltpu.VMEM((_CHUNK, _TOKEN_DIM), jnp.float32),
            pltpu.VMEM((_CHUNK, _CHAR_DIM), jnp.float32),
            pltpu.SemaphoreType.DMA,
        ],
        compiler_params=pltpu.CompilerParams(use_tc_tiling_on_sc=False),
    )(token_table, flat_idx, chars_pad)


_P = _B


def _tc_chars_body(cs_ref, len_ref, tbl_ref, out_ref):
    cs = cs_ref[0]
    ln = jnp.maximum(len_ref[0], 1)
    valid = lax.broadcasted_iota(jnp.int32, (_C, _P), 0) < ln
    cs_m = jnp.where(valid, cs, _CHAR_VOCAB)
    inv_len = 1.0 / ln.astype(jnp.float32)
    vocab = lax.broadcasted_iota(jnp.int32, (_CHAR_VOCAB, _P), 0)
    acc = jnp.zeros((_CHAR_VOCAB, _P), jnp.float32)
    for c in range(_C):
        acc = acc + (cs_m[c:c + 1, :] == vocab).astype(jnp.float32)
    acc = acc * inv_len
    chars = lax.dot_general(acc, tbl_ref[...], (((0,), (0,)), ((), ())),
                            preferred_element_type=jnp.float32)
    out_ref[:, 0:_CHAR_DIM] = chars


@jax.jit
def _tc_chars(cs3, ln3, char_table):
    return pl.pallas_call(
        _tc_chars_body,
        grid=(_L,),
        in_specs=[
            pl.BlockSpec((1, _C, _P), lambda i: (i, 0, 0)),
            pl.BlockSpec((1, 1, _P), lambda i: (i, 0, 0)),
            pl.BlockSpec((_CHAR_VOCAB, _CHAR_DIM), lambda i: (0, 0)),
        ],
        out_specs=pl.BlockSpec((_P, _PAD_DIM), lambda i: (i, 0)),
        out_shape=jax.ShapeDtypeStruct((_BL, _PAD_DIM), jnp.float32),
    )(cs3, ln3, char_table)


def kernel(token_seq, char_seq, char_lengths, token_table, char_table):
    flat_idx = token_seq.T.reshape(_BL).astype(jnp.int32)
    cs3 = jnp.transpose(char_seq, (1, 2, 0)).astype(jnp.int32)
    ln3 = char_lengths.T.reshape(_L, 1, _B).astype(jnp.int32)
    tbl_rm = lax.optimization_barrier(token_table.T).T
    chars_pad = _tc_chars(cs3, ln3, char_table)
    out_pad = _sc_merge(tbl_rm, flat_idx, chars_pad)
    out_lm = out_pad[:, :_OUT_DIM].reshape(_L, _B, _OUT_DIM)
    return jnp.transpose(out_lm, (1, 0, 2))
